# baseline (speedup 1.0000x reference)
import functools

import jax
import jax.numpy as jnp
from jax import lax
from jax.experimental import pallas as pl
from jax.experimental.pallas import tpu as pltpu
from jax.experimental.pallas import tpu_sc as plsc

NUM_CORES = 2
NUM_SUBCORES = 16
NUM_WORKERS = NUM_CORES * NUM_SUBCORES
NBUF = 2
L = 16
DSZ = 64
BLK = 128


@functools.lru_cache(maxsize=None)
def _make_lookup(hist: int, bsz: int, vhalf: int):
    n_units = hist * (bsz // BLK)
    assert n_units % NUM_WORKERS == 0
    u_per_w = n_units // NUM_WORKERS
    assert u_per_w % NBUF == 0
    n_outer = u_per_w // NBUF
    jb_per_row = bsz // BLK

    mesh = plsc.VectorSubcoreMesh(core_axis_name="c", subcore_axis_name="s")

    @functools.partial(
        pl.kernel,
        mesh=mesh,
        out_type=jax.ShapeDtypeStruct((hist, DSZ, bsz), jnp.float32),
        scratch_types=(
            [pltpu.VMEM((BLK,), jnp.int32) for _ in range(NBUF)]
            + [pltpu.VMEM((BLK,), jnp.int32) for _ in range(NBUF)]
            + [pltpu.VMEM((BLK,), jnp.int32) for _ in range(NBUF)]
            + [pltpu.VMEM((BLK, 2 * DSZ), jnp.float32) for _ in range(NBUF)]
            + [pltpu.VMEM((DSZ, BLK), jnp.float32) for _ in range(NBUF)]
            + [pltpu.SemaphoreType.DMA for _ in range(2 * NBUF)]
        ),
        compiler_params=pltpu.CompilerParams(
            use_tc_tiling_on_sc=True, needs_layout_passes=False),
    )
    def lookup(idx_hbm, table_hbm, out_hbm, *refs):
        ibuf = refs[:NBUF]
        wbuf = refs[NBUF:2 * NBUF]
        pbuf = refs[2 * NBUF:3 * NBUF]
        wide = refs[3 * NBUF:4 * NBUF]
        trans = refs[4 * NBUF:5 * NBUF]
        gsem = refs[5 * NBUF:6 * NBUF]
        ssem = refs[6 * NBUF:]
        wid = lax.axis_index("s") * NUM_CORES + lax.axis_index("c")
        ubase = wid * u_per_w

        def prep_idx(u, b):
            pltpu.sync_copy(idx_hbm.at[pl.ds(u * BLK, BLK)], ibuf[b])

            @pl.loop(0, BLK // L)
            def _g(g):
                v = ibuf[b][pl.ds(g * L, L)]
                wbuf[b][pl.ds(g * L, L)] = lax.shift_right_logical(v, 1)
                pbuf[b][pl.ds(g * L, L)] = (v & 1) * DSZ

        def start_gather(b):
            pltpu.async_copy(table_hbm.at[wbuf[b]], wide[b], gsem[b])

        def wait_gather(b):
            pltpu.make_async_copy(
                table_hbm.at[pl.ds(0, BLK)], wide[b], gsem[b]).wait()

        def transpose(b):
            for g in range(BLK // L):
                rv = lax.iota(jnp.int32, L) + g * L
                parv = pbuf[b][pl.ds(g * L, L)]

                @pl.loop(0, DSZ, unroll=8)
                def _d(d):
                    cv = parv + d
                    vals = plsc.load_gather(wide[b], [rv, cv])
                    trans[b][d, pl.ds(g * L, L)] = vals

        def start_store(u, b):
            h = u // jb_per_row
            jb = u % jb_per_row
            pltpu.async_copy(trans[b], out_hbm.at[h, :, pl.ds(jb * BLK, BLK)],
                             ssem[b])

        def wait_store(b):
            pltpu.make_async_copy(
                trans[b], out_hbm.at[0, :, pl.ds(0, BLK)], ssem[b]).wait()

        for b in range(NBUF):
            prep_idx(ubase + b, b)
            start_gather(b)

        @pl.loop(0, n_outer - 1)
        def _round(j):
            u0 = ubase + j * NBUF
            for b in range(NBUF):
                wait_gather(b)
                transpose(b)
                start_store(u0 + b, b)
            for b in range(NBUF):
                wait_store(b)
                prep_idx(u0 + NBUF + b, b)
                start_gather(b)

        u0 = ubase + (n_outer - 1) * NBUF
        for b in range(NBUF):
            wait_gather(b)
            transpose(b)
            start_store(u0 + b, b)
        for b in range(NBUF):
            wait_store(b)

    return lookup


def kernel(x, table):
    bsz, hist = x.shape
    vsz, dsz = table.shape
    flat_t = x.T.reshape(hist * bsz)
    wide_table = table.reshape(vsz // 2, 2 * dsz)
    lookup = _make_lookup(hist, bsz, vsz // 2)
    out3 = lookup(flat_t, wide_table)
    return out3.transpose(2, 0, 1)


# speedup vs baseline: 1.3286x; 1.3286x over previous
import functools

import jax
import jax.numpy as jnp
from jax import lax
from jax.experimental import pallas as pl
from jax.experimental.pallas import tpu as pltpu
from jax.experimental.pallas import tpu_sc as plsc

NUM_CORES = 2
NUM_SUBCORES = 16
NUM_WORKERS = NUM_CORES * NUM_SUBCORES
NBUF = 2
L = 16
DSZ = 64
BLK = 128


@functools.lru_cache(maxsize=None)
def _make_lookup(hist: int, bsz: int, vhalf: int):
    n_units = hist * (bsz // BLK)
    assert n_units % NUM_WORKERS == 0
    u_per_w = n_units // NUM_WORKERS
    assert u_per_w % NBUF == 0
    n_outer = u_per_w // NBUF
    jb_per_row = bsz // BLK

    mesh = plsc.VectorSubcoreMesh(core_axis_name="c", subcore_axis_name="s")

    @functools.partial(
        pl.kernel,
        mesh=mesh,
        out_type=jax.ShapeDtypeStruct((hist, DSZ, bsz), jnp.float32),
        scratch_types=(
            [pltpu.VMEM((BLK,), jnp.int32) for _ in range(NBUF)]
            + [pltpu.VMEM((BLK,), jnp.int32) for _ in range(NBUF)]
            + [pltpu.VMEM((BLK,), jnp.int32) for _ in range(NBUF)]
            + [pltpu.VMEM((BLK, 2 * DSZ), jnp.float32) for _ in range(NBUF)]
            + [pltpu.VMEM((DSZ, BLK), jnp.float32) for _ in range(NBUF)]
            + [pltpu.SemaphoreType.DMA for _ in range(2 * NBUF)]
        ),
        compiler_params=pltpu.CompilerParams(
            use_tc_tiling_on_sc=True, needs_layout_passes=False),
    )
    def lookup(idx_hbm, table_hbm, out_hbm, *refs):
        ibuf = refs[:NBUF]
        wbuf = refs[NBUF:2 * NBUF]
        pbuf = refs[2 * NBUF:3 * NBUF]
        wide = refs[3 * NBUF:4 * NBUF]
        trans = refs[4 * NBUF:5 * NBUF]
        gsem = refs[5 * NBUF:6 * NBUF]
        ssem = refs[6 * NBUF:]
        wid = lax.axis_index("s") * NUM_CORES + lax.axis_index("c")
        ubase = wid * u_per_w

        def prep_idx(u, b):
            pltpu.sync_copy(idx_hbm.at[pl.ds(u * BLK, BLK)], ibuf[b])

            @pl.loop(0, BLK // L)
            def _g(g):
                v = ibuf[b][pl.ds(g * L, L)]
                wbuf[b][pl.ds(g * L, L)] = lax.shift_right_logical(v, 1)
                pbuf[b][pl.ds(g * L, L)] = (v & 1) * DSZ

        def start_gather(b):
            pltpu.async_copy(table_hbm.at[wbuf[b]], wide[b], gsem[b])

        def wait_gather(b):
            pltpu.make_async_copy(
                table_hbm.at[pl.ds(0, BLK)], wide[b], gsem[b]).wait()

        def transpose(b):
            for g in range(BLK // L):
                rv = lax.iota(jnp.int32, L) + g * L
                parv = pbuf[b][pl.ds(g * L, L)]

                @pl.loop(0, DSZ // 8)
                def _d8(d8):
                    d0 = d8 * 8
                    cvs = [parv + (d0 + k) for k in range(8)]
                    vals = [plsc.load_gather(wide[b], [rv, cv]) for cv in cvs]
                    for k in range(8):
                        trans[b][d0 + k, pl.ds(g * L, L)] = vals[k]

        def start_store(u, b):
            h = u // jb_per_row
            jb = u % jb_per_row
            pltpu.async_copy(trans[b], out_hbm.at[h, :, pl.ds(jb * BLK, BLK)],
                             ssem[b])

        def wait_store(b):
            pltpu.make_async_copy(
                trans[b], out_hbm.at[0, :, pl.ds(0, BLK)], ssem[b]).wait()

        for b in range(NBUF):
            prep_idx(ubase + b, b)
            start_gather(b)

        @pl.loop(0, n_outer - 1)
        def _round(j):
            u0 = ubase + j * NBUF
            for b in range(NBUF):
                wait_gather(b)
                transpose(b)
                start_store(u0 + b, b)
            for b in range(NBUF):
                wait_store(b)
                prep_idx(u0 + NBUF + b, b)
                start_gather(b)

        u0 = ubase + (n_outer - 1) * NBUF
        for b in range(NBUF):
            wait_gather(b)
            transpose(b)
            start_store(u0 + b, b)
        for b in range(NBUF):
            wait_store(b)

    return lookup


def kernel(x, table):
    bsz, hist = x.shape
    vsz, dsz = table.shape
    flat_t = x.T.reshape(hist * bsz)
    wide_table = table.reshape(vsz // 2, 2 * dsz)
    lookup = _make_lookup(hist, bsz, vsz // 2)
    out3 = lookup(flat_t, wide_table)
    return out3.transpose(2, 0, 1)


# speedup vs baseline: 1.6442x; 1.2375x over previous
import functools

import jax
import jax.numpy as jnp
from jax import lax
from jax.experimental import pallas as pl
from jax.experimental.pallas import tpu as pltpu
from jax.experimental.pallas import tpu_sc as plsc

NUM_CORES = 2
NUM_SUBCORES = 16
NUM_WORKERS = NUM_CORES * NUM_SUBCORES
NBUF = 4


@functools.lru_cache(maxsize=None)
def _make_lookup(batch: int, vsz: int, dsz: int, chunk: int):
    assert batch % NUM_WORKERS == 0
    b_per_w = batch // NUM_WORKERS
    assert b_per_w % (chunk * NBUF) == 0
    n_outer = b_per_w // (chunk * NBUF)

    mesh = plsc.VectorSubcoreMesh(core_axis_name="c", subcore_axis_name="s")

    @functools.partial(
        pl.kernel,
        mesh=mesh,
        out_type=jax.ShapeDtypeStruct((batch, dsz), jnp.float32),
        scratch_types=(
            [pltpu.VMEM((b_per_w,), jnp.int32)]
            + [pltpu.VMEM((chunk, dsz), jnp.float32) for _ in range(NBUF)]
            + [pltpu.SemaphoreType.DMA for _ in range(2 * NBUF)]
        ),
        compiler_params=pltpu.CompilerParams(use_tc_tiling_on_sc=False),
    )
    def lookup(idx_hbm, table_hbm, out_hbm, idx_v, *bufs_and_sems):
        rows = bufs_and_sems[:NBUF]
        gsem = bufs_and_sems[NBUF:2 * NBUF]
        ssem = bufs_and_sems[2 * NBUF:]
        wid = lax.axis_index("s") * NUM_CORES + lax.axis_index("c")
        base = wid * b_per_w

        pltpu.sync_copy(idx_hbm.at[pl.ds(base, b_per_w)], idx_v)

        def start_gather(i, b):
            pltpu.async_copy(
                table_hbm.at[idx_v.at[pl.ds(i * chunk, chunk)]],
                rows[b], gsem[b])

        def start_store(i, b):
            pltpu.async_copy(rows[b], out_hbm.at[pl.ds(base + i * chunk, chunk)],
                             ssem[b])

        def wait_gather(b):
            pltpu.make_async_copy(
                out_hbm.at[pl.ds(0, chunk)], rows[b], gsem[b]).wait()

        def wait_store(b):
            pltpu.make_async_copy(
                rows[b], out_hbm.at[pl.ds(0, chunk)], ssem[b]).wait()

        for b in range(NBUF):
            start_gather(b, b)

        @pl.loop(0, n_outer - 1)
        def _round(j):
            i0 = j * NBUF
            for b in range(NBUF):
                wait_gather(b)
                start_store(i0 + b, b)
            for b in range(NBUF):
                wait_store(b)
                start_gather(i0 + NBUF + b, b)

        i0 = (n_outer - 1) * NBUF
        for b in range(NBUF):
            wait_gather(b)
            start_store(i0 + b, b)
        for b in range(NBUF):
            wait_store(b)

    return lookup


def kernel(x, table):
    bsz, hist = x.shape
    vsz, dsz = table.shape
    flat = x.reshape(bsz * hist)
    lookup = _make_lookup(bsz * hist, vsz, dsz, 256)
    out = lookup(flat, table)
    return out.reshape(bsz, hist, dsz)
